# baseline (device time: 8678 ns/iter reference)
import jax
import jax.numpy as jnp
from jax import lax
from jax.experimental import pallas as pl
from jax.experimental.pallas import tpu as pltpu


def kernel(x, gamma):
    m, n = x.shape
    n_global = 2 * n
    eps = 1e-5
    blk = 128
    n_blk = m // blk
    half = m // 2

    def body(x_hbm, gamma_ref, out_hbm, xv_ref, local_ref, peer_ref,
             in_sems, out_sems, send_sem, recv_sem):
        my_x = lax.axis_index("x")
        my_y = lax.axis_index("y")
        peer = (my_x, 1 - my_y)

        barrier_sem = pltpu.get_barrier_semaphore()
        pl.semaphore_signal(
            barrier_sem, inc=1, device_id=peer,
            device_id_type=pl.DeviceIdType.MESH,
        )

        cps = []
        for c in range(2):
            rows = pl.ds(c * half, half)
            cp = pltpu.make_async_copy(
                x_hbm.at[rows, :], xv_ref.at[rows, :], in_sems.at[c])
            cp.start()
            cps.append(cp)
        for c in range(2):
            cps[c].wait()
            rows = pl.ds(c * half, half)
            xc = xv_ref[rows, :]
            local_ref[pl.ds(c * (n_blk // 2), n_blk // 2), :] = (
                jnp.sum(xc * xc, axis=1).reshape(n_blk // 2, blk))

        pl.semaphore_wait(barrier_sem, 1)

        rdma = pltpu.make_async_remote_copy(
            src_ref=local_ref,
            dst_ref=peer_ref,
            send_sem=send_sem,
            recv_sem=recv_sem,
            device_id=peer,
            device_id_type=pl.DeviceIdType.MESH,
        )
        rdma.start()
        iota_r = lax.broadcasted_iota(jnp.int32, (blk, blk), 0)
        iota_c = lax.broadcasted_iota(jnp.int32, (blk, blk), 1)
        eye = (iota_r == iota_c).astype(jnp.float32)
        gv = gamma_ref[...]
        rdma.wait()

        total = local_ref[...] + peer_ref[...]
        scale = lax.rsqrt(total / n_global + eps)
        out_cps = []
        for i in range(n_blk):
            lane_i = scale[i:i + 1, :]
            col_i = jnp.sum(eye * lane_i, axis=1, keepdims=True)
            rows = pl.ds(i * blk, blk)
            xv_ref[rows, :] = gv * xv_ref[rows, :] * col_i
            cp = pltpu.make_async_copy(
                xv_ref.at[rows, :], out_hbm.at[rows, :], out_sems.at[i])
            cp.start()
            out_cps.append(cp)
        for cp in out_cps:
            cp.wait()

    return pl.pallas_call(
        body,
        out_shape=jax.ShapeDtypeStruct((m, n), jnp.float32),
        in_specs=[
            pl.BlockSpec(memory_space=pl.ANY),
            pl.BlockSpec(memory_space=pltpu.VMEM),
        ],
        out_specs=pl.BlockSpec(memory_space=pl.ANY),
        scratch_shapes=[
            pltpu.VMEM((m, n), jnp.float32),
            pltpu.VMEM((n_blk, blk), jnp.float32),
            pltpu.VMEM((n_blk, blk), jnp.float32),
            pltpu.SemaphoreType.DMA((2,)),
            pltpu.SemaphoreType.DMA((n_blk,)),
            pltpu.SemaphoreType.DMA,
            pltpu.SemaphoreType.DMA,
        ],
        compiler_params=pltpu.CompilerParams(collective_id=0),
    )(x, gamma.reshape(1, n))


# device time: 7908 ns/iter; 1.0974x vs baseline; 1.0974x over previous
import jax
import jax.numpy as jnp
from jax import lax
from jax.experimental import pallas as pl
from jax.experimental.pallas import tpu as pltpu


def kernel(x, gamma):
    m, n = x.shape
    n_global = 2 * n
    eps = 1e-5
    blk = 128
    n_blk = m // blk
    half = m // 2

    def body(x_hbm, gamma_ref, out_ref, xv_ref, local_ref, peer_ref,
             in_sems, out_sems, send_sem, recv_sem):
        my_x = lax.axis_index("x")
        my_y = lax.axis_index("y")
        peer = (my_x, 1 - my_y)

        barrier_sem = pltpu.get_barrier_semaphore()
        pl.semaphore_signal(
            barrier_sem, inc=1, device_id=peer,
            device_id_type=pl.DeviceIdType.MESH,
        )

        cp = pltpu.make_async_copy(x_hbm, xv_ref, in_sems.at[0])
        cp.start()
        cp.wait()
        xv = xv_ref[...]
        local_ref[...] = jnp.sum(xv * xv, axis=1).reshape(n_blk, blk)

        pl.semaphore_wait(barrier_sem, 1)

        rdma = pltpu.make_async_remote_copy(
            src_ref=local_ref,
            dst_ref=peer_ref,
            send_sem=send_sem,
            recv_sem=recv_sem,
            device_id=peer,
            device_id_type=pl.DeviceIdType.MESH,
        )
        rdma.start()
        iota_r = lax.broadcasted_iota(jnp.int32, (blk, blk), 0)
        iota_c = lax.broadcasted_iota(jnp.int32, (blk, blk), 1)
        eye = (iota_r == iota_c).astype(jnp.float32)
        gv = gamma_ref[...]
        rdma.wait()

        total = local_ref[...] + peer_ref[...]
        scale = lax.rsqrt(total / n_global + eps)
        for i in range(n_blk):
            lane_i = scale[i:i + 1, :]
            col_i = jnp.sum(eye * lane_i, axis=1, keepdims=True)
            rows = pl.ds(i * blk, blk)
            out_ref[rows, :] = gv * xv_ref[rows, :] * col_i

    return pl.pallas_call(
        body,
        out_shape=jax.ShapeDtypeStruct((m, n), jnp.float32),
        in_specs=[
            pl.BlockSpec(memory_space=pl.ANY),
            pl.BlockSpec(memory_space=pltpu.VMEM),
        ],
        out_specs=pl.BlockSpec(memory_space=pltpu.VMEM),
        scratch_shapes=[
            pltpu.VMEM((m, n), jnp.float32),
            pltpu.VMEM((n_blk, blk), jnp.float32),
            pltpu.VMEM((n_blk, blk), jnp.float32),
            pltpu.SemaphoreType.DMA((2,)),
            pltpu.SemaphoreType.DMA((n_blk,)),
            pltpu.SemaphoreType.DMA,
            pltpu.SemaphoreType.DMA,
        ],
        compiler_params=pltpu.CompilerParams(collective_id=0),
    )(x, gamma.reshape(1, n))


# device time: 7717 ns/iter; 1.1245x vs baseline; 1.0248x over previous
import jax
import jax.numpy as jnp
from jax import lax
from jax.experimental import pallas as pl
from jax.experimental.pallas import tpu as pltpu


def kernel(x, gamma):
    m, n = x.shape
    n_global = 2 * n
    eps = 1e-5

    blk = 128
    n_blk = m // blk

    def body(x_ref, gamma_ref, out_ref, local_ref, peer_ref, send_sem, recv_sem):
        my_x = lax.axis_index("x")
        my_y = lax.axis_index("y")
        peer = (my_x, 1 - my_y)

        barrier_sem = pltpu.get_barrier_semaphore()
        pl.semaphore_signal(
            barrier_sem, inc=1, device_id=peer,
            device_id_type=pl.DeviceIdType.MESH,
        )

        xv = x_ref[...]
        local_ref[...] = jnp.sum(xv * xv, axis=1).reshape(n_blk, blk)

        pl.semaphore_wait(barrier_sem, 1)

        rdma = pltpu.make_async_remote_copy(
            src_ref=local_ref,
            dst_ref=peer_ref,
            send_sem=send_sem,
            recv_sem=recv_sem,
            device_id=peer,
            device_id_type=pl.DeviceIdType.MESH,
        )
        rdma.start()
        iota_r = lax.broadcasted_iota(jnp.int32, (blk, blk), 0)
        iota_c = lax.broadcasted_iota(jnp.int32, (blk, blk), 1)
        eye = (iota_r == iota_c).astype(jnp.float32)
        gv = gamma_ref[...]
        rdma.wait()

        total = local_ref[...] + peer_ref[...]
        scale = lax.rsqrt(total / n_global + eps)
        for i in range(n_blk):
            lane_i = scale[i:i + 1, :]
            col_i = jnp.sum(eye * lane_i, axis=1, keepdims=True)
            rows = pl.ds(i * blk, blk)
            out_ref[rows, :] = gv * x_ref[rows, :] * col_i

    return pl.pallas_call(
        body,
        out_shape=jax.ShapeDtypeStruct((m, n), jnp.float32),
        in_specs=[
            pl.BlockSpec(memory_space=pltpu.VMEM),
            pl.BlockSpec(memory_space=pltpu.VMEM),
        ],
        out_specs=pl.BlockSpec(memory_space=pltpu.VMEM),
        scratch_shapes=[
            pltpu.VMEM((m // 128, 128), jnp.float32),
            pltpu.VMEM((m // 128, 128), jnp.float32),
            pltpu.SemaphoreType.DMA,
            pltpu.SemaphoreType.DMA,
        ],
        compiler_params=pltpu.CompilerParams(collective_id=0),
    )(x, gamma.reshape(1, n))
